# baseline (speedup 1.0000x reference)
import functools

import jax
import jax.numpy as jnp
from jax import lax
from jax.experimental import pallas as pl
from jax.experimental.pallas import tpu as pltpu
from jax.experimental.pallas import tpu_sc as plsc

B, T, D = 32, 500, 3000
N = B * T
NW = 32
NJ = 4
GW = 128
PER_W = NJ * GW
NPAD = NW * PER_W
LANES = 16


def _sc_body(x_hbm, tgt_hbm, out_hbm, tgt_v, idx_v, val_v, acc_v, sem):
    cid = lax.axis_index("c")
    sid = lax.axis_index("s")
    wid = sid * 2 + cid
    base = wid * PER_W

    pltpu.sync_copy(tgt_hbm.at[wid], tgt_v)

    lane = lax.iota(jnp.int32, LANES)
    for j in range(NJ):
        for c in range(GW // LANES):
            t16 = tgt_v[j, pl.ds(c * LANES, LANES)]
            row = base + j * GW + c * LANES + lane
            idx16 = row * D + t16
            idx16 = jnp.where(row < N, idx16, 0)
            idx_v[j, pl.ds(c * LANES, LANES)] = idx16

    copies = [
        pltpu.async_copy(x_hbm.at[idx_v.at[j]], val_v.at[j], sem)
        for j in range(NJ)
    ]
    for cp in copies:
        cp.wait()

    acc = jnp.zeros((LANES,), jnp.float32)
    for j in range(NJ):
        for c in range(GW // LANES):
            v16 = val_v[j, pl.ds(c * LANES, LANES)]
            v16 = jnp.clip(v16, -30.0, 30.0)
            row = base + j * GW + c * LANES + lane
            acc = acc + jnp.where(row < N, v16, 0.0)

    acc_v[...] = acc * (-1.0 / N)
    pltpu.sync_copy(acc_v, out_hbm.at[wid])


@jax.jit
def _chain_loss(xflat, tgt):
    mesh = plsc.VectorSubcoreMesh(core_axis_name="c", subcore_axis_name="s")
    partials = pl.kernel(
        _sc_body,
        mesh=mesh,
        out_type=jax.ShapeDtypeStruct((NW, LANES), jnp.float32),
        scratch_types=[
            pltpu.VMEM((NJ, GW), jnp.int32),
            pltpu.VMEM((NJ, GW), jnp.int32),
            pltpu.VMEM((NJ, GW), jnp.float32),
            pltpu.VMEM((LANES,), jnp.float32),
            pltpu.SemaphoreType.DMA,
        ],
    )(xflat, tgt)
    return jnp.sum(partials)


def kernel(x, targets):
    tgt = targets.reshape(-1).astype(jnp.int32)
    tgt = jnp.pad(tgt, (0, NPAD - N))
    tgt = tgt.reshape(NW, NJ, GW)
    xflat = x.reshape(-1)
    return _chain_loss(xflat, tgt)


# speedup vs baseline: 12.7296x; 12.7296x over previous
import jax
import jax.numpy as jnp
from jax import lax
from jax.experimental import pallas as pl
from jax.experimental.pallas import tpu as pltpu
from jax.experimental.pallas import tpu_sc as plsc

B, T, D = 32, 500, 3000
N = B * T
NW = 32
LANES = 16
TPAD = 512
NCH = TPAD // LANES
SUB, LN = 8, 128


def _sc_body(x_hbm, tgt_hbm, out_hbm, tgt_v, buf, acc_v, sem0, sem1):
    cid = lax.axis_index("c")
    sid = lax.axis_index("s")
    wid = sid * 2 + cid

    pltpu.sync_copy(tgt_hbm.at[wid], tgt_v)

    sems = [sem0, sem1]

    def chunk_targets(c):
        return tgt_v[c // SUB, pl.ds((c % SUB) * LANES, LANES)]

    def enqueue(c, parity):
        col16 = chunk_targets(c) & ~(LN - 1)
        for j in range(LANES):
            i = c * LANES + j
            row0 = pl.multiple_of(i & ~(SUB - 1), SUB)
            col0 = pl.multiple_of(col16[j], LN)
            pltpu.async_copy(
                x_hbm.at[wid, pl.ds(row0, SUB), pl.ds(col0, LN)],
                buf.at[parity, j],
                sems[parity],
            )

    def drain(parity):
        pltpu.make_async_copy(
            x_hbm.at[pl.ds(0, LANES), pl.ds(0, SUB), pl.ds(0, LN)],
            buf.at[parity],
            sems[parity],
        ).wait()

    lane = lax.iota(jnp.int32, LANES)
    sub_idx = lane & (SUB - 1)

    def extract(c, parity, acc):
        t16 = chunk_targets(c)
        v16 = plsc.load_gather(buf.at[parity], [lane, sub_idx, t16 & (LN - 1)])
        v16 = jnp.clip(v16, -30.0, 30.0)
        v16 = jnp.where(c * LANES + lane < T, v16, 0.0)
        return acc + v16

    acc = jnp.zeros((LANES,), jnp.float32)
    enqueue(0, 0)
    for c in range(NCH):
        if c + 1 < NCH:
            enqueue(c + 1, (c + 1) % 2)
        drain(c % 2)
        acc = extract(c, c % 2, acc)

    acc_v[...] = acc * (-1.0 / N)
    pltpu.sync_copy(acc_v, out_hbm.at[wid])


@jax.jit
def _chain_loss(x, tgt):
    mesh = plsc.VectorSubcoreMesh(core_axis_name="c", subcore_axis_name="s")
    partials = pl.kernel(
        _sc_body,
        mesh=mesh,
        compiler_params=pltpu.CompilerParams(needs_layout_passes=False),
        out_type=jax.ShapeDtypeStruct((NW, LANES), jnp.float32),
        scratch_types=[
            pltpu.VMEM((TPAD // LN, LN), jnp.int32),
            pltpu.VMEM((2, LANES, SUB, LN), jnp.float32),
            pltpu.VMEM((LANES,), jnp.float32),
            pltpu.SemaphoreType.DMA,
            pltpu.SemaphoreType.DMA,
        ],
    )(x, tgt)
    return jnp.sum(partials)


def kernel(x, targets):
    tgt = targets.astype(jnp.int32)
    tgt = jnp.pad(tgt, ((0, 0), (0, TPAD - T)))
    tgt = tgt.reshape(NW, TPAD // LN, LN)
    return _chain_loss(x, tgt)
